# baseline (speedup 1.0000x reference)
import numpy as np

import jax
import jax.numpy as jnp
from jax import lax
from jax.experimental import pallas as pl
from jax.experimental.pallas import tpu as pltpu
from jax.experimental.pallas import tpu_sc as plsc

NC = 2
NS = 16
LANES = 16
NW = NC * NS

B = 4096
H = 50
D = 64
BPW = B // NW
ROWS2 = BPW // 2
EPS = 1e-5

K = 10
NGRP = (H + K - 1) // K

_PERM = np.concatenate([np.arange(0, 32, 2), np.arange(1, 32, 2),
                        np.arange(32, 64, 2), np.arange(33, 64, 2)])


def _grp_hs(g):
    return range(g * K, min((g + 1) * K, H))


def _pool_body(xt_hbm, user_hbm, genre_hbm, out_hbm,
               idx_v, bufs, acc, sem0, sem1):
    wid = lax.axis_index("s") * NC + lax.axis_index("c")
    sems = (sem0, sem1)
    mask = jnp.broadcast_to(jnp.uint32(0xFFFF0000), (LANES,))
    sh16 = jnp.broadcast_to(jnp.uint32(16), (LANES,))

    for t, table in ((0, user_hbm), (1, genre_hbm)):
        pltpu.sync_copy(xt_hbm.at[t, :, wid], idx_v)

        def _issue(g):
            bank = g % 2
            for k, h in enumerate(_grp_hs(g)):
                pltpu.async_copy(table.at[idx_v.at[h]], bufs.at[bank, k],
                                 sems[bank])

        def _drain(g):
            bank = g % 2
            for k, h in enumerate(_grp_hs(g)):
                pltpu.make_async_copy(table.at[idx_v.at[h]], bufs.at[bank, k],
                                      sems[bank]).wait()

        def _accum(g):
            bank = g % 2
            nk = len(_grp_hs(g))

            @plsc.parallel_loop(0, ROWS2, unroll=2)
            def body(p):
                for half in range(2):
                    i = 2 * p + half
                    for c in range(2):
                        s_e = s_o = None
                        for k in range(nk):
                            bv = bufs[bank, k, i, pl.ds(c * 32, 32)]
                            u = plsc.bitcast(bv, jnp.uint32)
                            lo = plsc.bitcast(u << sh16, jnp.float32)
                            hi = plsc.bitcast(u & mask, jnp.float32)
                            s_e = lo if s_e is None else s_e + lo
                            s_o = hi if s_o is None else s_o + hi
                        base = half * 64 + c * 32
                        if g > 0:
                            s_e = s_e + acc[p, pl.ds(base, LANES)]
                            s_o = s_o + acc[p, pl.ds(base + LANES, LANES)]
                        acc[p, pl.ds(base, LANES)] = s_e
                        acc[p, pl.ds(base + LANES, LANES)] = s_o

        _issue(0)
        _issue(1)
        for g in range(NGRP):
            _drain(g)
            _accum(g)
            if g + 2 < NGRP:
                _issue(g + 2)

        pltpu.sync_copy(acc, out_hbm.at[t, pl.ds(wid * ROWS2, ROWS2)])


def _pool(xt, user_table, genre_table):
    mesh = plsc.VectorSubcoreMesh(core_axis_name="c", subcore_axis_name="s",
                                  num_cores=NC, num_subcores=NS)
    return pl.kernel(
        _pool_body,
        out_type=jax.ShapeDtypeStruct((2, B // 2, 128), jnp.float32),
        mesh=mesh,
        scratch_types=[
            pltpu.VMEM((H, BPW), jnp.int32),
            pltpu.VMEM((2, K, BPW, D), jnp.bfloat16),
            pltpu.VMEM((ROWS2, 128), jnp.float32),
            pltpu.SemaphoreType.DMA,
            pltpu.SemaphoreType.DMA,
        ],
        compiler_params=pltpu.CompilerParams(use_tc_tiling_on_sc=False,
                                             needs_layout_passes=False),
    )(xt, user_table, genre_table)


def _bn_dot_body(emb_ref, gamma_ref, beta_ref, out_ref):
    gamma = gamma_ref[...]
    beta = beta_ref[...]

    def bn(h):
        n = 2.0 / B
        m = jnp.sum(h, axis=0, keepdims=True) * n
        sq = jnp.sum(h * h, axis=0, keepdims=True) * n
        mu = (m[:, :D] + m[:, D:]) * 0.5
        var = (sq[:, :D] + sq[:, D:]) * 0.5 - mu * mu
        a = gamma * lax.rsqrt(var + EPS)
        b = beta - a * mu
        a2 = jnp.concatenate([a, a], axis=1)
        b2 = jnp.concatenate([b, b], axis=1)
        return h * a2 + b2

    u = bn(emb_ref[0] * (1.0 / H))
    g = bn(emb_ref[1] * (1.0 / H))
    prod = u * g
    z0 = jnp.sum(prod[:, :D], axis=1, keepdims=True)
    z1 = jnp.sum(prod[:, D:], axis=1, keepdims=True)
    out_ref[...] = jax.nn.sigmoid(jnp.concatenate([z0, z1], axis=1))


def _bn_dot(pooled, gamma, beta):
    return pl.pallas_call(
        _bn_dot_body,
        out_shape=jax.ShapeDtypeStruct((B // 2, 2), jnp.float32),
    )(pooled, gamma, beta)


def kernel(x, user_table, genre_table, gamma, beta):
    xt = jnp.transpose(x.astype(jnp.int32), (0, 2, 1)).reshape(2, H, NW, 128)
    ut16 = user_table.astype(jnp.bfloat16)
    gt16 = genre_table.astype(jnp.bfloat16)
    pooled = _pool(xt, ut16, gt16)
    perm = jnp.asarray(_PERM)
    z = _bn_dot(pooled, gamma[perm].reshape(1, D), beta[perm].reshape(1, D))
    return z.reshape(B)
